# baseline (speedup 1.0000x reference)
import functools

import jax
import jax.numpy as jnp
from jax.experimental import pallas as pl
from jax.experimental.pallas import tpu as pltpu

LANE = 128
_VMEM = pl.BlockSpec(memory_space=pltpu.MemorySpace.VMEM)


def _cp(**kw):
    return pltpu.CompilerParams(vmem_limit_bytes=64 * 1024 * 1024, **kw)


def _feat_kernel(x_ref, w1_ref, b1_ref, w2_ref, b2_ref, m_ref,
                 pad1_ref, pad2_ref):
    H, W, CIN = x_ref.shape
    HF, WF = H // 2, W // 2
    CP = LANE

    pad1_ref[...] = jnp.zeros(pad1_ref.shape, jnp.float32)
    pad1_ref[1:H + 1, 1:W + 1, :] = x_ref[...]

    acc = jnp.zeros((H * W, CP), jnp.float32)
    for ky in range(3):
        for kx in range(3):
            patch = pad1_ref[ky:ky + H, kx:kx + W, :].reshape(H * W, CIN)
            acc = acc + jnp.dot(patch, w1_ref[ky * 3 + kx],
                                preferred_element_type=jnp.float32)
    h1 = jnp.maximum(acc + b1_ref[...], 0.0).reshape(H, W, CP)

    h1 = jnp.max(h1.reshape(HF, 2, W, CP), axis=1)
    h1 = jnp.max(h1.reshape(HF, WF, 2, CP), axis=2)

    pad2_ref[...] = jnp.zeros(pad2_ref.shape, jnp.float32)
    pad2_ref[1:HF + 1, 1:WF + 1, :] = h1
    cols = [pad2_ref[ky:ky + HF, kx:kx + WF, :].reshape(HF * WF, CP)
            for ky in range(3) for kx in range(3)]
    col = jnp.concatenate(cols, axis=-1)
    f = jnp.dot(col, w2_ref[...], preferred_element_type=jnp.float32)
    f = jnp.maximum(f + b2_ref[...], 0.0).reshape(HF, WF, CP)

    m2 = jnp.maximum(f, jnp.concatenate([f[1:], f[HF - 1:]], axis=0))
    m3 = jnp.maximum(m2, jnp.concatenate([f[2:], f[HF - 2:]], axis=0))
    m4 = jnp.maximum(m3, jnp.concatenate([f[3:], f[HF - 3:]], axis=0))
    m_ref[0:HF] = f
    m_ref[HF:2 * HF] = m2
    m_ref[2 * HF:3 * HF] = m3
    m_ref[3 * HF:4 * HF] = m4


def _features(x_hwc, c1w, c1b, c2w, c2b):
    H, W, CIN = x_hwc.shape
    HF, WF = H // 2, W // 2
    flops = 2 * (H * W * 9 * CIN * LANE + HF * WF * 9 * LANE * LANE)
    bytes_acc = 4 * (H * W * CIN + 9 * CIN * LANE + 9 * LANE * LANE
                     + 4 * HF * WF * LANE)
    return pl.pallas_call(
        _feat_kernel,
        out_shape=jax.ShapeDtypeStruct((4 * HF, WF, LANE), jnp.float32),
        in_specs=[_VMEM] * 5,
        out_specs=_VMEM,
        scratch_shapes=[pltpu.VMEM((H + 2, W + 2, CIN), jnp.float32),
                        pltpu.VMEM((HF + 2, WF + 2, LANE), jnp.float32)],
        compiler_params=_cp(),
        cost_estimate=pl.CostEstimate(flops=flops, transcendentals=0,
                                      bytes_accessed=bytes_acc),
    )(x_hwc, c1w, c1b, c2w, c2b)


def _roi_kernel(idx_ref, m_ref, o_ref, *, pool, batch, win):
    P = pool
    step = pl.program_id(0)
    cols = jax.lax.broadcasted_iota(jnp.int32, (1, win, 1), 1)
    vals = []
    for b in range(batch):
        r = step * batch + b
        w0 = idx_ref[r, 0]
        w1 = idx_ref[r, 1]
        w2 = idx_ref[r, 2]
        w3 = idx_ref[r, 3]
        x1 = w0 & 31
        ridx = [(w0 >> (5 + 7 * oy)) & 127 for oy in range(3)] + \
               [(w1 >> (7 * (oy - 3))) & 127 for oy in range(3, P)]
        u0s = [(w2 >> (5 * ox)) & 31 for ox in range(6)] + [w3 & 31]
        u1s = [u0s[ox] + 1 + ((w3 >> (5 + 2 * ox)) & 3) for ox in range(P)]
        rows = [m_ref[pl.ds(ridx[oy], 1), pl.ds(x1, win), :]
                for oy in range(P)]
        rb = jnp.concatenate(rows, axis=0)
        vals.append([])
        for ox in range(P):
            cm = (cols >= u0s[ox]) & (cols < u1s[ox])
            v = jnp.max(jnp.where(cm, rb, 0.0), axis=1)
            vals[b].append(v.astype(o_ref.dtype))
    for ox in range(P):
        stacked = jnp.stack([vals[b][ox] for b in range(batch)],
                            axis=1)
        for oy in range(P):
            o_ref[oy * P + ox] = stacked[oy]


def _roi_pool(m_table, idx, pool_size, batch, win):
    Rp = idx.shape[0]
    MH, WF, _ = m_table.shape
    kfn = functools.partial(_roi_kernel, pool=pool_size, batch=batch, win=win)
    grid_spec = pltpu.PrefetchScalarGridSpec(
        num_scalar_prefetch=1,
        grid=(Rp // batch,),
        in_specs=[pl.BlockSpec((MH, WF, LANE), lambda s, idx: (0, 0, 0))],
        out_specs=pl.BlockSpec((pool_size * pool_size, batch, LANE),
                               lambda s, idx: (0, s, 0)),
    )
    return pl.pallas_call(
        kfn,
        out_shape=jax.ShapeDtypeStruct((pool_size * pool_size, Rp, LANE),
                                       jnp.bfloat16),
        grid_spec=grid_spec,
        compiler_params=_cp(dimension_semantics=("parallel",)),
    )(idx, m_table)


def _bin_tables(rois, hf, wf, pool_size, win):
    P = pool_size
    x1 = jnp.clip(rois[:, 0], 0, wf - win)
    y1 = jnp.clip(rois[:, 1], 0, hf - 1)
    w = jnp.clip(rois[:, 2] - x1, 1, win)
    h = jnp.clip(rois[:, 3] - y1, 1, win)
    o = jnp.arange(P, dtype=jnp.int32)[None, :]
    t0 = (o * h[:, None]) // P
    t1 = ((o + 1) * h[:, None] + P - 1) // P
    ridx = (t1 - t0 - 1) * hf + y1[:, None] + t0
    u0 = (o * w[:, None]) // P
    u1 = ((o + 1) * w[:, None] + P - 1) // P
    sm1 = u1 - u0 - 1
    w0 = x1
    for oy in range(3):
        w0 = w0 | (ridx[:, oy] << (5 + 7 * oy))
    w1 = ridx[:, 3]
    for oy in range(4, P):
        w1 = w1 | (ridx[:, oy] << (7 * (oy - 3)))
    w2 = u0[:, 0]
    for ox in range(1, 6):
        w2 = w2 | (u0[:, ox] << (5 * ox))
    w3 = u0[:, 6]
    for ox in range(P):
        w3 = w3 | (sm1[:, ox] << (5 + 2 * ox))
    return jnp.stack([w0, w1, w2, w3], axis=1)


def _fc1_kernel(x3_ref, w1_ref, b1_ref, h_ref, acc_ref, *, nk):
    k = pl.program_id(1)
    KB = x3_ref.shape[0]
    acc = jnp.dot(x3_ref[0], w1_ref[0:LANE, :].astype(jnp.bfloat16),
                  preferred_element_type=jnp.float32)
    for j in range(1, KB):
        wj = w1_ref[j * LANE:(j + 1) * LANE, :].astype(jnp.bfloat16)
        acc = acc + jnp.dot(x3_ref[j], wj,
                            preferred_element_type=jnp.float32)

    @pl.when(k == 0)
    def _init():
        acc_ref[...] = acc

    @pl.when(k > 0)
    def _accum():
        acc_ref[...] = acc_ref[...] + acc

    @pl.when(k == nk - 1)
    def _fin():
        h_ref[...] = jnp.maximum(acc_ref[...] + b1_ref[...], 0.0)


def _fc1(x3, w1, b1, n_blocks, nk):
    NB, Rp, _ = x3.shape
    D = w1.shape[0]
    H1 = w1.shape[1]
    HB = H1 // n_blocks
    KB = NB // nk
    flops = 2 * Rp * D * H1
    bytes_acc = 2 * n_blocks * Rp * D + 4 * (D * H1 + Rp * H1)
    return pl.pallas_call(
        functools.partial(_fc1_kernel, nk=nk),
        out_shape=jax.ShapeDtypeStruct((Rp, H1), jnp.float32),
        grid=(n_blocks, nk),
        in_specs=[pl.BlockSpec((KB, Rp, LANE), lambda j, k: (k, 0, 0)),
                  pl.BlockSpec((KB * LANE, HB), lambda j, k: (k, j)),
                  pl.BlockSpec((1, HB), lambda j, k: (0, j))],
        out_specs=pl.BlockSpec((Rp, HB), lambda j, k: (0, j)),
        scratch_shapes=[pltpu.VMEM((Rp, HB), jnp.float32)],
        compiler_params=_cp(dimension_semantics=("parallel", "arbitrary")),
        cost_estimate=pl.CostEstimate(flops=flops, transcendentals=0,
                                      bytes_accessed=bytes_acc),
    )(x3, w1, b1)


def _fc23_kernel(h_ref, w23_ref, b23_ref, o_ref):
    o_ref[...] = (jnp.dot(h_ref[...], w23_ref[...],
                          preferred_element_type=jnp.float32) + b23_ref[...])


def _fc23(h, w23, b23, n_blocks):
    Rp, H1 = h.shape
    RB = Rp // n_blocks
    return pl.pallas_call(
        _fc23_kernel,
        out_shape=jax.ShapeDtypeStruct((Rp, LANE), jnp.float32),
        grid=(n_blocks,),
        in_specs=[pl.BlockSpec((RB, H1), lambda j: (j, 0)),
                  pl.BlockSpec((H1, LANE), lambda j: (0, 0)),
                  pl.BlockSpec((1, LANE), lambda j: (0, 0))],
        out_specs=pl.BlockSpec((RB, LANE), lambda j: (j, 0)),
        compiler_params=_cp(dimension_semantics=("parallel",)),
    )(h, w23, b23)


def kernel(images, rois, conv1_w_p, conv1_b_p, conv2_w_p, conv2_b_p,
           fc1_w_p, fc1_b_p, head_w_p, head_b_p, pool_size=7, num_classes=21):
    x = jnp.transpose(images, (0, 2, 3, 1))[0]
    H, W, _ = x.shape
    HF, WF = H // 2, W // 2
    P = pool_size
    WIN = WF // 2

    m_table = _features(x, conv1_w_p, conv1_b_p, conv2_w_p, conv2_b_p)

    R = rois.shape[0]
    Rp = max(8, ((R + 7) // 8) * 8)
    rois_p = jnp.pad(rois, ((0, Rp - R), (0, 0)))
    idx = _bin_tables(rois_p, HF, WF, P, WIN)

    pooled = _roi_pool(m_table, idx, P, 8, WIN)

    h = _fc1(pooled, fc1_w_p, fc1_b_p, n_blocks=2, nk=7)
    out = _fc23(h, head_w_p, head_b_p, n_blocks=2)
    cls_scores = out[:R, :num_classes]
    bbox_preds = out[:R, num_classes:num_classes + 4]
    return cls_scores, bbox_preds


# speedup vs baseline: 1.0262x; 1.0262x over previous
import functools

import jax
import jax.numpy as jnp
from jax.experimental import pallas as pl
from jax.experimental.pallas import tpu as pltpu

LANE = 128
_VMEM = pl.BlockSpec(memory_space=pltpu.MemorySpace.VMEM)


def _cp(**kw):
    return pltpu.CompilerParams(vmem_limit_bytes=64 * 1024 * 1024, **kw)


def _feat_kernel(x_ref, w1_ref, b1_ref, w2_ref, b2_ref, m_ref,
                 pad1_ref, pad2_ref):
    H, W, CIN = x_ref.shape
    HF, WF = H // 2, W // 2
    CP = LANE

    pad1_ref[...] = jnp.zeros(pad1_ref.shape, jnp.float32)
    pad1_ref[1:H + 1, 1:W + 1, :] = x_ref[...]

    acc = jnp.zeros((H * W, CP), jnp.float32)
    for ky in range(3):
        for kx in range(3):
            patch = pad1_ref[ky:ky + H, kx:kx + W, :].reshape(H * W, CIN)
            acc = acc + jnp.dot(patch, w1_ref[ky * 3 + kx],
                                preferred_element_type=jnp.float32)
    h1 = jnp.maximum(acc + b1_ref[...], 0.0).reshape(H, W, CP)

    h1 = jnp.max(h1.reshape(HF, 2, W, CP), axis=1)
    h1 = jnp.max(h1.reshape(HF, WF, 2, CP), axis=2)

    pad2_ref[...] = jnp.zeros(pad2_ref.shape, jnp.float32)
    pad2_ref[1:HF + 1, 1:WF + 1, :] = h1
    cols = [pad2_ref[ky:ky + HF, kx:kx + WF, :].reshape(HF * WF, CP)
            for ky in range(3) for kx in range(3)]
    col = jnp.concatenate(cols, axis=-1)
    f = jnp.dot(col, w2_ref[...], preferred_element_type=jnp.float32)
    f = jnp.maximum(f + b2_ref[...], 0.0).reshape(HF, WF, CP)

    m2 = jnp.maximum(f, jnp.concatenate([f[1:], f[HF - 1:]], axis=0))
    m3 = jnp.maximum(m2, jnp.concatenate([f[2:], f[HF - 2:]], axis=0))
    m4 = jnp.maximum(m3, jnp.concatenate([f[3:], f[HF - 3:]], axis=0))
    m_ref[0:HF] = f
    m_ref[HF:2 * HF] = m2
    m_ref[2 * HF:3 * HF] = m3
    m_ref[3 * HF:4 * HF] = m4


def _features(x_hwc, c1w, c1b, c2w, c2b):
    H, W, CIN = x_hwc.shape
    HF, WF = H // 2, W // 2
    flops = 2 * (H * W * 9 * CIN * LANE + HF * WF * 9 * LANE * LANE)
    bytes_acc = 4 * (H * W * CIN + 9 * CIN * LANE + 9 * LANE * LANE
                     + 4 * HF * WF * LANE)
    return pl.pallas_call(
        _feat_kernel,
        out_shape=jax.ShapeDtypeStruct((4 * HF, WF, LANE), jnp.float32),
        in_specs=[_VMEM] * 5,
        out_specs=_VMEM,
        scratch_shapes=[pltpu.VMEM((H + 2, W + 2, CIN), jnp.float32),
                        pltpu.VMEM((HF + 2, WF + 2, LANE), jnp.float32)],
        compiler_params=_cp(),
        cost_estimate=pl.CostEstimate(flops=flops, transcendentals=0,
                                      bytes_accessed=bytes_acc),
    )(x_hwc, c1w, c1b, c2w, c2b)


def _roi_kernel(idx_ref, m_ref, o_ref, *, pool, batch, win):
    P = pool
    step = pl.program_id(0)
    cols = jax.lax.broadcasted_iota(jnp.int32, (1, win, 1), 1)
    vals = []
    for b in range(batch):
        r = step * batch + b
        w0 = idx_ref[r, 0]
        w1 = idx_ref[r, 1]
        w2 = idx_ref[r, 2]
        w3 = idx_ref[r, 3]
        x1 = w0 & 31
        ridx = [(w0 >> (5 + 7 * oy)) & 127 for oy in range(3)] + \
               [(w1 >> (7 * (oy - 3))) & 127 for oy in range(3, P)]
        u0s = [(w2 >> (5 * ox)) & 31 for ox in range(6)] + [w3 & 31]
        u1s = [u0s[ox] + 1 + ((w3 >> (5 + 2 * ox)) & 3) for ox in range(P)]
        rows = [m_ref[pl.ds(ridx[oy], 1), pl.ds(x1, win), :]
                for oy in range(P)]
        rb = jnp.concatenate(rows, axis=0)
        vals.append([])
        for ox in range(P):
            cm = (cols >= u0s[ox]) & (cols < u1s[ox])
            vals[b].append(jnp.max(jnp.where(cm, rb, 0.0), axis=1))
    for ox in range(P):
        stacked = jnp.stack([vals[b][ox] for b in range(batch)],
                            axis=1).astype(o_ref.dtype)
        for oy in range(P):
            o_ref[oy * P + ox] = stacked[oy]


def _roi_pool(m_table, idx, pool_size, batch, win):
    Rp = idx.shape[0]
    MH, WF, _ = m_table.shape
    kfn = functools.partial(_roi_kernel, pool=pool_size, batch=batch, win=win)
    grid_spec = pltpu.PrefetchScalarGridSpec(
        num_scalar_prefetch=1,
        grid=(Rp // batch,),
        in_specs=[pl.BlockSpec((MH, WF, LANE), lambda s, idx: (0, 0, 0))],
        out_specs=pl.BlockSpec((pool_size * pool_size, batch, LANE),
                               lambda s, idx: (0, s, 0)),
    )
    return pl.pallas_call(
        kfn,
        out_shape=jax.ShapeDtypeStruct((pool_size * pool_size, Rp, LANE),
                                       jnp.bfloat16),
        grid_spec=grid_spec,
        compiler_params=_cp(dimension_semantics=("parallel",)),
    )(idx, m_table)


def _bin_tables(rois, hf, wf, pool_size, win):
    P = pool_size
    x1 = jnp.clip(rois[:, 0], 0, wf - win)
    y1 = jnp.clip(rois[:, 1], 0, hf - 1)
    w = jnp.clip(rois[:, 2] - x1, 1, win)
    h = jnp.clip(rois[:, 3] - y1, 1, win)
    o = jnp.arange(P, dtype=jnp.int32)[None, :]
    t0 = (o * h[:, None]) // P
    t1 = ((o + 1) * h[:, None] + P - 1) // P
    ridx = (t1 - t0 - 1) * hf + y1[:, None] + t0
    u0 = (o * w[:, None]) // P
    u1 = ((o + 1) * w[:, None] + P - 1) // P
    sm1 = u1 - u0 - 1
    w0 = x1
    for oy in range(3):
        w0 = w0 | (ridx[:, oy] << (5 + 7 * oy))
    w1 = ridx[:, 3]
    for oy in range(4, P):
        w1 = w1 | (ridx[:, oy] << (7 * (oy - 3)))
    w2 = u0[:, 0]
    for ox in range(1, 6):
        w2 = w2 | (u0[:, ox] << (5 * ox))
    w3 = u0[:, 6]
    for ox in range(P):
        w3 = w3 | (sm1[:, ox] << (5 + 2 * ox))
    return jnp.stack([w0, w1, w2, w3], axis=1)


def _fc1_kernel(x3_ref, w1_ref, b1_ref, h_ref, acc_ref, *, nk):
    k = pl.program_id(1)
    KB = x3_ref.shape[0]
    acc = jnp.dot(x3_ref[0], w1_ref[0:LANE, :].astype(jnp.bfloat16),
                  preferred_element_type=jnp.float32)
    for j in range(1, KB):
        wj = w1_ref[j * LANE:(j + 1) * LANE, :].astype(jnp.bfloat16)
        acc = acc + jnp.dot(x3_ref[j], wj,
                            preferred_element_type=jnp.float32)

    @pl.when(k == 0)
    def _init():
        acc_ref[...] = acc

    @pl.when(k > 0)
    def _accum():
        acc_ref[...] = acc_ref[...] + acc

    @pl.when(k == nk - 1)
    def _fin():
        h_ref[...] = jnp.maximum(acc_ref[...] + b1_ref[...], 0.0)


def _fc1(x3, w1, b1, n_blocks, nk):
    NB, Rp, _ = x3.shape
    D = w1.shape[0]
    H1 = w1.shape[1]
    HB = H1 // n_blocks
    KB = NB // nk
    flops = 2 * Rp * D * H1
    bytes_acc = 2 * n_blocks * Rp * D + 4 * (D * H1 + Rp * H1)
    return pl.pallas_call(
        functools.partial(_fc1_kernel, nk=nk),
        out_shape=jax.ShapeDtypeStruct((Rp, H1), jnp.float32),
        grid=(n_blocks, nk),
        in_specs=[pl.BlockSpec((KB, Rp, LANE), lambda j, k: (k, 0, 0)),
                  pl.BlockSpec((KB * LANE, HB), lambda j, k: (k, j)),
                  pl.BlockSpec((1, HB), lambda j, k: (0, j))],
        out_specs=pl.BlockSpec((Rp, HB), lambda j, k: (0, j)),
        scratch_shapes=[pltpu.VMEM((Rp, HB), jnp.float32)],
        compiler_params=_cp(dimension_semantics=("parallel", "arbitrary")),
        cost_estimate=pl.CostEstimate(flops=flops, transcendentals=0,
                                      bytes_accessed=bytes_acc),
    )(x3, w1, b1)


def _fc23_kernel(h_ref, w23_ref, b23_ref, o_ref):
    o_ref[...] = (jnp.dot(h_ref[...], w23_ref[...],
                          preferred_element_type=jnp.float32) + b23_ref[...])


def _fc23(h, w23, b23, n_blocks):
    Rp, H1 = h.shape
    RB = Rp // n_blocks
    return pl.pallas_call(
        _fc23_kernel,
        out_shape=jax.ShapeDtypeStruct((Rp, LANE), jnp.float32),
        grid=(n_blocks,),
        in_specs=[pl.BlockSpec((RB, H1), lambda j: (j, 0)),
                  pl.BlockSpec((H1, LANE), lambda j: (0, 0)),
                  pl.BlockSpec((1, LANE), lambda j: (0, 0))],
        out_specs=pl.BlockSpec((RB, LANE), lambda j: (j, 0)),
        compiler_params=_cp(dimension_semantics=("parallel",)),
    )(h, w23, b23)


def kernel(images, rois, conv1_w_p, conv1_b_p, conv2_w_p, conv2_b_p,
           fc1_w_p, fc1_b_p, head_w_p, head_b_p, pool_size=7, num_classes=21):
    x = jnp.transpose(images, (0, 2, 3, 1))[0]
    H, W, _ = x.shape
    HF, WF = H // 2, W // 2
    P = pool_size
    WIN = WF // 2

    m_table = _features(x, conv1_w_p, conv1_b_p, conv2_w_p, conv2_b_p)

    R = rois.shape[0]
    Rp = max(8, ((R + 7) // 8) * 8)
    rois_p = jnp.pad(rois, ((0, Rp - R), (0, 0)))
    idx = _bin_tables(rois_p, HF, WF, P, WIN)

    pooled = _roi_pool(m_table, idx, P, 8, WIN)

    h = _fc1(pooled, fc1_w_p, fc1_b_p, n_blocks=2, nk=7)
    out = _fc23(h, head_w_p, head_b_p, n_blocks=2)
    cls_scores = out[:R, :num_classes]
    bbox_preds = out[:R, num_classes:num_classes + 4]
    return cls_scores, bbox_preds
